# baseline (speedup 1.0000x reference)
import functools

import jax
import jax.numpy as jnp
from jax import lax
from jax.experimental import pallas as pl
from jax.experimental.pallas import tpu as pltpu
from jax.experimental.pallas import tpu_sc as plsc

_B, _T, _D = 4, 4096, 1024
_A0, _A1 = 64, 64
_NC, _NS = 2, 16
_NW = _NC * _NS
_IPW = _A0 // _NW
_HJ = _A1 // 2


def _sc_body(w0_hbm, w1_hbm, out_hbm, w0_v, w1_v, obuf, osem):
    wid = lax.axis_index("s") * _NC + lax.axis_index("c")
    pltpu.sync_copy(w0_hbm.at[pl.ds(wid * _IPW, _IPW), :], w0_v)

    def compute(slot, ii):
        def dloop(dc, _):
            va = w0_v[ii, pl.ds(dc * 16, 16)]
            for j in range(_HJ):
                obuf[slot, j, pl.ds(dc * 16, 16)] = (
                    va + w1_v[j, pl.ds(dc * 16, 16)])
            return 0
        lax.fori_loop(0, _D // 16, dloop, 0)

    def unit_t0(u):
        return (wid * _IPW + (u % _IPW)) * _A1 + (u // _IPW) * _HJ

    for h in range(2):
        pltpu.sync_copy(w1_hbm.at[pl.ds(h * _HJ, _HJ), :], w1_v)
        for ii in range(_IPW):
            u = h * _IPW + ii
            slot = u % 2
            if u >= 2:
                pt0 = unit_t0(u - 2)
                for b in range(_B):
                    pltpu.make_async_copy(
                        obuf.at[slot],
                        out_hbm.at[pl.ds(b * _T + pt0, _HJ), :],
                        osem.at[slot, b],
                    ).wait()
            compute(slot, ii)
            t0 = unit_t0(u)
            for b in range(_B):
                pltpu.make_async_copy(
                    obuf.at[slot],
                    out_hbm.at[pl.ds(b * _T + t0, _HJ), :],
                    osem.at[slot, b],
                ).start()

    for u in (2, 3):
        slot = u % 2
        t0 = unit_t0(u)
        for b in range(_B):
            pltpu.make_async_copy(
                obuf.at[slot],
                out_hbm.at[pl.ds(b * _T + t0, _HJ), :],
                osem.at[slot, b],
            ).wait()


_sc_call = functools.partial(
    pl.kernel,
    mesh=plsc.VectorSubcoreMesh(core_axis_name="c", subcore_axis_name="s"),
    out_type=jax.ShapeDtypeStruct((_B * _T, _D), jnp.float32),
    scratch_types=[
        pltpu.VMEM((_IPW, _D), jnp.float32),
        pltpu.VMEM((_HJ, _D), jnp.float32),
        pltpu.VMEM((2, _HJ, _D), jnp.float32),
        pltpu.SemaphoreType.DMA((2, _B)),
    ],
)(_sc_body)


def kernel(x, w0, w1):
    del x
    out = _sc_call(w0.reshape(_A0, _D), w1.reshape(_A1, _D))
    return out.reshape(_B, _T, _D)
